# baseline (speedup 1.0000x reference)
import functools

import jax
import jax.numpy as jnp
from jax import lax
from jax.experimental import pallas as pl
from jax.experimental.pallas import tpu as pltpu
from jax.experimental.pallas import tpu_sc as plsc

N = 10000
D = 128
E = 320000
NP = 10240
EP = 327680
NW = 32
CPT = 80
CH = 128
RPT = NP // 16
HR = NP // 16
BLK = 1024

_mesh = plsc.VectorSubcoreMesh(core_axis_name="c", subcore_axis_name="s")
_sc_params = pltpu.CompilerParams(needs_layout_passes=False)


@functools.partial(
    pl.kernel,
    mesh=_mesh,
    out_type=jax.ShapeDtypeStruct((2, NP), jnp.float32),
    scratch_types=[
        pltpu.VMEM_SHARED((16, NP), jnp.float32),
        pltpu.VMEM((NP,), jnp.float32),
        pltpu.VMEM((CPT, CH), jnp.int32),
        pltpu.VMEM((16, NP // 16), jnp.float32),
        pltpu.VMEM((NP // 16,), jnp.float32),
    ],
    compiler_params=_sc_params,
)
def _deg_hist(dst_hbm, out_hbm, hist_sh, hist_v, dst_v, red_v, out_v):
    c = lax.axis_index("c")
    s = lax.axis_index("s")
    wid = c * 16 + s
    zz = jnp.zeros((16,), jnp.float32)

    def _zh(i, _):
        hist_v[pl.ds(i * 16, 16)] = zz
        return 0

    lax.fori_loop(0, NP // 16, _zh, 0)
    pltpu.sync_copy(dst_hbm.at[pl.ds(wid * CPT, CPT)], dst_v)

    ones = jnp.ones((16,), jnp.float32)

    def _acc(i, _):
        for j in range(8):
            dv = dst_v[i, pl.ds(j * 16, 16)]
            plsc.addupdate_scatter(hist_v, [dv], ones)
        return 0

    lax.fori_loop(0, CPT, _acc, 0)
    pltpu.sync_copy(hist_v, hist_sh.at[s])
    plsc.subcore_barrier()
    pltpu.sync_copy(hist_sh.at[:, pl.ds(s * (NP // 16), NP // 16)], red_v)

    def _red(cc, _):
        acc = red_v[0, pl.ds(cc * 16, 16)]
        for r in range(1, 16):
            acc = acc + red_v[r, pl.ds(cc * 16, 16)]
        out_v[pl.ds(cc * 16, 16)] = acc
        return 0

    lax.fori_loop(0, NP // 256, _red, 0)
    pltpu.sync_copy(out_v, out_hbm.at[c, pl.ds(s * (NP // 16), NP // 16)])


NB = 5
ECH = 32
EPH = 40
C0PH = 8
C1PH = 8


@functools.partial(
    pl.kernel,
    mesh=_mesh,
    out_type=jax.ShapeDtypeStruct((2, NP, D), jnp.float32),
    scratch_types=[
        pltpu.VMEM_SHARED((NP, D), jnp.float32),
        pltpu.VMEM((NB, ECH, D), jnp.float32),
        pltpu.VMEM((EPH, ECH), jnp.int32),
        pltpu.VMEM((EPH, ECH), jnp.int32),
        pltpu.SemaphoreType.DMA,
        pltpu.SemaphoreType.DMA,
        pltpu.SemaphoreType.DMA,
        pltpu.SemaphoreType.DMA,
        pltpu.SemaphoreType.DMA,
        pltpu.SemaphoreType.DMA,
        pltpu.SemaphoreType.DMA,
        pltpu.SemaphoreType.DMA,
        pltpu.SemaphoreType.DMA,
        pltpu.SemaphoreType.DMA,
    ],
    compiler_params=_sc_params,
)
def _edge_agg(g_hbm, src_hbm, dst_hbm, out_hbm, acc_sh, rows_v, src_v, dst_v,
              g0, g1, g2, g3, g4, s0, s1, s2, s3, s4):
    c = lax.axis_index("c")
    s = lax.axis_index("s")
    gs = (g0, g1, g2, g3, g4)
    ss = (s0, s1, s2, s3, s4)
    n_ph = jnp.where(c == 0, C0PH, C1PH)
    chunk0 = jnp.where(c == 0, s * (C0PH * EPH),
                       16 * C0PH * EPH + s * (C1PH * EPH))
    zz = jnp.zeros((16,), jnp.float32)

    def _zrow(i, _):
        for j in range(D // 16):
            rows_v[0, i, pl.ds(j * 16, 16)] = zz
        return 0

    lax.fori_loop(0, ECH, _zrow, 0)
    for k in range(RPT // ECH):
        pltpu.sync_copy(rows_v.at[0], acc_sh.at[pl.ds(s * RPT + k * ECH, ECH)])
    plsc.subcore_barrier()

    def _phase(p, _):
        off = chunk0 + p * EPH
        pltpu.sync_copy(src_hbm.at[pl.ds(off, EPH)], src_v)
        pltpu.sync_copy(dst_hbm.at[pl.ds(off, EPH)], dst_v)
        for b in range(NB):
            pltpu.async_copy(g_hbm.at[src_v.at[b]], rows_v.at[b], gs[b])

        def _step(ip, _):
            base = ip * NB
            for b in range(NB):
                ic = base + b
                pltpu.make_async_copy(g_hbm.at[src_v.at[ic]], rows_v.at[b],
                                      gs[b]).wait()
                pltpu.async_copy(rows_v.at[b], acc_sh.at[dst_v.at[ic]], ss[b],
                                 add=True)
            for b in range(NB):
                pltpu.make_async_copy(rows_v.at[b],
                                      acc_sh.at[dst_v.at[base + b]],
                                      ss[b]).wait()
                pltpu.async_copy(g_hbm.at[src_v.at[base + NB + b]],
                                 rows_v.at[b], gs[b])
            return 0

        lax.fori_loop(0, EPH // NB - 1, _step, 0)
        base = EPH - NB
        for b in range(NB):
            ic = base + b
            pltpu.make_async_copy(g_hbm.at[src_v.at[ic]], rows_v.at[b],
                                  gs[b]).wait()
            pltpu.async_copy(rows_v.at[b], acc_sh.at[dst_v.at[ic]], ss[b],
                             add=True)
        for b in range(NB):
            pltpu.make_async_copy(rows_v.at[b], acc_sh.at[dst_v.at[base + b]],
                                  ss[b]).wait()
        return 0

    lax.fori_loop(0, n_ph, _phase, 0)
    plsc.subcore_barrier()
    for k in range(RPT // ECH):
        r0 = s * RPT + k * ECH
        pltpu.sync_copy(acc_sh.at[pl.ds(r0, ECH)], out_hbm.at[c, pl.ds(r0, ECH)])


def _t1_body(x_ref, w_ref, degt_ref, g_ref, dis_ref):
    d = degt_ref[...]
    dis = lax.rsqrt(d[:, 0:1] + d[:, 1:2] + 1.0)
    g_ref[...] = jnp.dot(x_ref[...], w_ref[...],
                         preferred_element_type=jnp.float32) * dis
    dis_ref[...] = dis


def _t2_body(p_ref, g1_ref, dis_ref, b1_ref, w2_ref, g2_ref):
    dis = dis_ref[...]
    z = jnp.maximum((p_ref[0] + p_ref[1] + g1_ref[...]) * dis + b1_ref[...], 0.0)
    g2_ref[...] = jnp.dot(z, w2_ref[...],
                          preferred_element_type=jnp.float32) * dis


def _t3_body(p_ref, g2_ref, dis_ref, b2_ref, o_ref):
    o_ref[...] = (p_ref[0] + p_ref[1] + g2_ref[...]) * dis_ref[...] + b2_ref[...]


def _t1(xp, W1, degt):
    return pl.pallas_call(
        _t1_body,
        grid=(NP // BLK,),
        in_specs=[
            pl.BlockSpec((BLK, D), lambda i: (i, 0)),
            pl.BlockSpec((D, D), lambda i: (0, 0)),
            pl.BlockSpec((BLK, 2), lambda i: (i, 0)),
        ],
        out_specs=[
            pl.BlockSpec((BLK, D), lambda i: (i, 0)),
            pl.BlockSpec((BLK, 1), lambda i: (i, 0)),
        ],
        out_shape=[
            jax.ShapeDtypeStruct((NP, D), jnp.float32),
            jax.ShapeDtypeStruct((NP, 1), jnp.float32),
        ],
    )(xp, W1, degt)


def _t2(p1, g1, dis, b1, W2):
    return pl.pallas_call(
        _t2_body,
        grid=(NP // BLK,),
        in_specs=[
            pl.BlockSpec((2, BLK, D), lambda i: (0, i, 0)),
            pl.BlockSpec((BLK, D), lambda i: (i, 0)),
            pl.BlockSpec((BLK, 1), lambda i: (i, 0)),
            pl.BlockSpec((1, D), lambda i: (0, 0)),
            pl.BlockSpec((D, D), lambda i: (0, 0)),
        ],
        out_specs=pl.BlockSpec((BLK, D), lambda i: (i, 0)),
        out_shape=jax.ShapeDtypeStruct((NP, D), jnp.float32),
    )(p1, g1, dis, b1, W2)


def _t3(p2, g2, dis, b2):
    return pl.pallas_call(
        _t3_body,
        grid=(NP // BLK,),
        in_specs=[
            pl.BlockSpec((2, BLK, D), lambda i: (0, i, 0)),
            pl.BlockSpec((BLK, D), lambda i: (i, 0)),
            pl.BlockSpec((BLK, 1), lambda i: (i, 0)),
            pl.BlockSpec((1, D), lambda i: (0, 0)),
        ],
        out_specs=pl.BlockSpec((BLK, D), lambda i: (i, 0)),
        out_shape=jax.ShapeDtypeStruct((NP, D), jnp.float32),
    )(p2, g2, dis, b2)


def kernel(x, edge_index, W1, b1, W2, b2):
    ei = edge_index.astype(jnp.int32)
    fill = N + jnp.arange(EP - E, dtype=jnp.int32) % (NP - N)
    src = jnp.concatenate([ei[0], fill]).reshape(EP // CH, CH)
    dst = jnp.concatenate([ei[1], fill]).reshape(EP // CH, CH)
    xp = jnp.pad(x, ((0, NP - N), (0, 0)))

    src_e = src.reshape(EP // ECH, ECH)
    dst_e = dst.reshape(EP // ECH, ECH)

    degp = _deg_hist(dst)
    degt = degp.T
    g1, dis = _t1(xp, W1, degt)
    p1 = _edge_agg(g1, src_e, dst_e)
    g2 = _t2(p1, g1, dis, b1.reshape(1, D), W2)
    p2 = _edge_agg(g2, src_e, dst_e)
    out = _t3(p2, g2, dis, b2.reshape(1, D))
    return out[:N]


# speedup vs baseline: 1.1248x; 1.1248x over previous
import functools

import jax
import jax.numpy as jnp
from jax import lax
from jax.experimental import pallas as pl
from jax.experimental.pallas import tpu as pltpu
from jax.experimental.pallas import tpu_sc as plsc

N = 10000
D = 128
E = 320000
NP = 10240
EP = 327680
NW = 32
CPT = 80
CH = 128
RPT = NP // 16
HR = NP // 16
BLK = 1024

_mesh = plsc.VectorSubcoreMesh(core_axis_name="c", subcore_axis_name="s")
_sc_params = pltpu.CompilerParams(needs_layout_passes=False)


@functools.partial(
    pl.kernel,
    mesh=_mesh,
    out_type=jax.ShapeDtypeStruct((2, NP), jnp.float32),
    scratch_types=[
        pltpu.VMEM_SHARED((16, NP), jnp.float32),
        pltpu.VMEM((NP,), jnp.float32),
        pltpu.VMEM((CPT, CH), jnp.int32),
        pltpu.VMEM((16, NP // 16), jnp.float32),
        pltpu.VMEM((NP // 16,), jnp.float32),
    ],
    compiler_params=_sc_params,
)
def _deg_hist(dst_hbm, out_hbm, hist_sh, hist_v, dst_v, red_v, out_v):
    c = lax.axis_index("c")
    s = lax.axis_index("s")
    wid = c * 16 + s
    zz = jnp.zeros((16,), jnp.float32)

    def _zh(i, _):
        hist_v[pl.ds(i * 16, 16)] = zz
        return 0

    lax.fori_loop(0, NP // 16, _zh, 0)
    pltpu.sync_copy(dst_hbm.at[pl.ds(wid * CPT, CPT)], dst_v)

    ones = jnp.ones((16,), jnp.float32)

    def _acc(i, _):
        for j in range(8):
            dv = dst_v[i, pl.ds(j * 16, 16)]
            plsc.addupdate_scatter(hist_v, [dv], ones)
        return 0

    lax.fori_loop(0, CPT, _acc, 0)
    pltpu.sync_copy(hist_v, hist_sh.at[s])
    plsc.subcore_barrier()
    pltpu.sync_copy(hist_sh.at[:, pl.ds(s * (NP // 16), NP // 16)], red_v)

    def _red(cc, _):
        acc = red_v[0, pl.ds(cc * 16, 16)]
        for r in range(1, 16):
            acc = acc + red_v[r, pl.ds(cc * 16, 16)]
        out_v[pl.ds(cc * 16, 16)] = acc
        return 0

    lax.fori_loop(0, NP // 256, _red, 0)
    pltpu.sync_copy(out_v, out_hbm.at[c, pl.ds(s * (NP // 16), NP // 16)])


NB = 4
ECH = 80
EPH = 32
C0PH = 4
C1PH = 4


@functools.partial(
    pl.kernel,
    mesh=_mesh,
    out_type=jax.ShapeDtypeStruct((2, NP, D), jnp.float32),
    scratch_types=[
        pltpu.VMEM_SHARED((NP, D), jnp.float32),
        pltpu.VMEM((NB, ECH, D), jnp.float32),
        pltpu.VMEM((EPH, ECH), jnp.int32),
        pltpu.VMEM((EPH, ECH), jnp.int32),
        pltpu.SemaphoreType.DMA,
        pltpu.SemaphoreType.DMA,
        pltpu.SemaphoreType.DMA,
        pltpu.SemaphoreType.DMA,
        pltpu.SemaphoreType.DMA,
        pltpu.SemaphoreType.DMA,
        pltpu.SemaphoreType.DMA,
        pltpu.SemaphoreType.DMA,
    ],
    compiler_params=_sc_params,
)
def _edge_agg(g_hbm, src_hbm, dst_hbm, out_hbm, acc_sh, rows_v, src_v, dst_v,
              g0, g1, g2, g3, s0, s1, s2, s3):
    c = lax.axis_index("c")
    s = lax.axis_index("s")
    gs = (g0, g1, g2, g3)
    ss = (s0, s1, s2, s3)
    n_ph = jnp.where(c == 0, C0PH, C1PH)
    chunk0 = jnp.where(c == 0, s * (C0PH * EPH),
                       16 * C0PH * EPH + s * (C1PH * EPH))
    zz = jnp.zeros((16,), jnp.float32)

    def _zrow(i, _):
        for j in range(D // 16):
            rows_v[0, i, pl.ds(j * 16, 16)] = zz
        return 0

    lax.fori_loop(0, ECH, _zrow, 0)
    for k in range(RPT // ECH):
        pltpu.sync_copy(rows_v.at[0], acc_sh.at[pl.ds(s * RPT + k * ECH, ECH)])
    plsc.subcore_barrier()

    def _phase(p, _):
        off = chunk0 + p * EPH
        pltpu.sync_copy(src_hbm.at[pl.ds(off, EPH)], src_v)
        pltpu.sync_copy(dst_hbm.at[pl.ds(off, EPH)], dst_v)
        for b in range(NB):
            pltpu.async_copy(g_hbm.at[src_v.at[b]], rows_v.at[b], gs[b])

        def _step(ip, _):
            base = ip * NB
            for b in range(NB):
                ic = base + b
                pltpu.make_async_copy(g_hbm.at[src_v.at[ic]], rows_v.at[b],
                                      gs[b]).wait()
                pltpu.async_copy(rows_v.at[b], acc_sh.at[dst_v.at[ic]], ss[b],
                                 add=True)
            for b in range(NB):
                pltpu.make_async_copy(rows_v.at[b],
                                      acc_sh.at[dst_v.at[base + b]],
                                      ss[b]).wait()
                pltpu.async_copy(g_hbm.at[src_v.at[base + NB + b]],
                                 rows_v.at[b], gs[b])
            return 0

        lax.fori_loop(0, EPH // NB - 1, _step, 0)
        base = EPH - NB
        for b in range(NB):
            ic = base + b
            pltpu.make_async_copy(g_hbm.at[src_v.at[ic]], rows_v.at[b],
                                  gs[b]).wait()
            pltpu.async_copy(rows_v.at[b], acc_sh.at[dst_v.at[ic]], ss[b],
                             add=True)
        for b in range(NB):
            pltpu.make_async_copy(rows_v.at[b], acc_sh.at[dst_v.at[base + b]],
                                  ss[b]).wait()
        return 0

    lax.fori_loop(0, n_ph, _phase, 0)
    plsc.subcore_barrier()
    for k in range(RPT // ECH):
        r0 = s * RPT + k * ECH
        pltpu.sync_copy(acc_sh.at[pl.ds(r0, ECH)], out_hbm.at[c, pl.ds(r0, ECH)])


def _t1_body(x_ref, w_ref, degt_ref, g_ref, dis_ref):
    d = degt_ref[...]
    dis = lax.rsqrt(d[:, 0:1] + d[:, 1:2] + 1.0)
    g_ref[...] = jnp.dot(x_ref[...], w_ref[...],
                         preferred_element_type=jnp.float32) * dis
    dis_ref[...] = dis


def _t2_body(p_ref, g1_ref, dis_ref, b1_ref, w2_ref, g2_ref):
    dis = dis_ref[...]
    z = jnp.maximum((p_ref[0] + p_ref[1] + g1_ref[...]) * dis + b1_ref[...], 0.0)
    g2_ref[...] = jnp.dot(z, w2_ref[...],
                          preferred_element_type=jnp.float32) * dis


def _t3_body(p_ref, g2_ref, dis_ref, b2_ref, o_ref):
    o_ref[...] = (p_ref[0] + p_ref[1] + g2_ref[...]) * dis_ref[...] + b2_ref[...]


def _t1(xp, W1, degt):
    return pl.pallas_call(
        _t1_body,
        grid=(NP // BLK,),
        in_specs=[
            pl.BlockSpec((BLK, D), lambda i: (i, 0)),
            pl.BlockSpec((D, D), lambda i: (0, 0)),
            pl.BlockSpec((BLK, 2), lambda i: (i, 0)),
        ],
        out_specs=[
            pl.BlockSpec((BLK, D), lambda i: (i, 0)),
            pl.BlockSpec((BLK, 1), lambda i: (i, 0)),
        ],
        out_shape=[
            jax.ShapeDtypeStruct((NP, D), jnp.float32),
            jax.ShapeDtypeStruct((NP, 1), jnp.float32),
        ],
    )(xp, W1, degt)


def _t2(p1, g1, dis, b1, W2):
    return pl.pallas_call(
        _t2_body,
        grid=(NP // BLK,),
        in_specs=[
            pl.BlockSpec((2, BLK, D), lambda i: (0, i, 0)),
            pl.BlockSpec((BLK, D), lambda i: (i, 0)),
            pl.BlockSpec((BLK, 1), lambda i: (i, 0)),
            pl.BlockSpec((1, D), lambda i: (0, 0)),
            pl.BlockSpec((D, D), lambda i: (0, 0)),
        ],
        out_specs=pl.BlockSpec((BLK, D), lambda i: (i, 0)),
        out_shape=jax.ShapeDtypeStruct((NP, D), jnp.float32),
    )(p1, g1, dis, b1, W2)


def _t3(p2, g2, dis, b2):
    return pl.pallas_call(
        _t3_body,
        grid=(NP // BLK,),
        in_specs=[
            pl.BlockSpec((2, BLK, D), lambda i: (0, i, 0)),
            pl.BlockSpec((BLK, D), lambda i: (i, 0)),
            pl.BlockSpec((BLK, 1), lambda i: (i, 0)),
            pl.BlockSpec((1, D), lambda i: (0, 0)),
        ],
        out_specs=pl.BlockSpec((BLK, D), lambda i: (i, 0)),
        out_shape=jax.ShapeDtypeStruct((NP, D), jnp.float32),
    )(p2, g2, dis, b2)


def kernel(x, edge_index, W1, b1, W2, b2):
    ei = edge_index.astype(jnp.int32)
    fill = N + jnp.arange(EP - E, dtype=jnp.int32) % (NP - N)
    src = jnp.concatenate([ei[0], fill]).reshape(EP // CH, CH)
    dst = jnp.concatenate([ei[1], fill]).reshape(EP // CH, CH)
    xp = jnp.pad(x, ((0, NP - N), (0, 0)))

    src_e = src.reshape(EP // ECH, ECH)
    dst_e = dst.reshape(EP // ECH, ECH)

    degp = _deg_hist(dst)
    degt = degp.T
    g1, dis = _t1(xp, W1, degt)
    p1 = _edge_agg(g1, src_e, dst_e)
    g2 = _t2(p1, g1, dis, b1.reshape(1, D), W2)
    p2 = _edge_agg(g2, src_e, dst_e)
    out = _t3(p2, g2, dis, b2.reshape(1, D))
    return out[:N]
